# baseline (speedup 1.0000x reference)
import functools

import jax
import jax.numpy as jnp
from jax import lax
from jax.experimental import pallas as pl
from jax.experimental.pallas import tpu as pltpu
from jax.experimental.pallas import tpu_sc as plsc

_NC = 2
_NS = 16
_NW = _NC * _NS


def _emb_gather(V, D, N, F, chn):
    n_per_w = N // _NW
    npair = n_per_w // (2 * chn)
    rpl = 128 // D
    mesh = plsc.VectorSubcoreMesh(core_axis_name="c", subcore_axis_name="s")

    @functools.partial(
        pl.kernel,
        mesh=mesh,
        out_type=jax.ShapeDtypeStruct((N, F, D), jnp.float32),
        compiler_params=pltpu.CompilerParams(use_tc_tiling_on_sc=True),
        scratch_types=[
            pltpu.VMEM((chn, F), jnp.int32),
            pltpu.VMEM((chn, F), jnp.int32),
            pltpu.VMEM((chn, F), jnp.int32),
            pltpu.VMEM((chn, F), jnp.int32),
            pltpu.VMEM((chn, F, 128), jnp.float32),
            pltpu.VMEM((chn, F, 128), jnp.float32),
            pltpu.VMEM((chn, F, D), jnp.float32),
            pltpu.VMEM((chn, F, D), jnp.float32),
            pltpu.SemaphoreType.DMA,
            pltpu.SemaphoreType.DMA,
            pltpu.SemaphoreType.DMA,
            pltpu.SemaphoreType.DMA,
        ],
    )
    def body(table128, idx4, r4, out,
             ia, ib, ra, rb, rowsa, rowsb, oa, ob, ga, gb, wa, wb):
        wid = lax.axis_index("s") * _NC + lax.axis_index("c")
        base = wid * n_per_w

        def stage_and_fire(nb, idx_v, r_v, rows_v, gsem):
            pltpu.sync_copy(idx4.at[pl.ds(nb, chn)], idx_v)
            pltpu.sync_copy(r4.at[pl.ds(nb, chn)], r_v)
            return [
                pltpu.async_copy(
                    table128.at[idx_v.at[a]], rows_v.at[a], gsem)
                for a in range(chn)
            ]

        def extract(r_v, rows_v, o_v):
            for a in range(chn):
                rv0 = r_v[a, pl.ds(0, 16)]
                rv1 = r_v[a, pl.ds(F - 16, 16)]
                for f in range(F):
                    r = rv0[f] if f < 16 else rv1[f - (F - 16)]
                    col = r * D
                    o_v[a, f, pl.ds(0, 16)] = rows_v[a, f, pl.ds(col, 16)]
                    o_v[a, f, pl.ds(16, 16)] = rows_v[
                        a, f, pl.ds(col + 16, 16)]

        def pair(c2, _):
            nb0 = base + c2 * (2 * chn)
            nb1 = nb0 + chn
            g0 = stage_and_fire(nb0, ia, ra, rowsa, ga)
            g1 = stage_and_fire(nb1, ib, rb, rowsb, gb)
            for g in g0:
                g.wait()
            extract(ra, rowsa, oa)
            w0 = pltpu.async_copy(oa, out.at[pl.ds(nb0, chn)], wa)
            for g in g1:
                g.wait()
            extract(rb, rowsb, ob)
            w1 = pltpu.async_copy(ob, out.at[pl.ds(nb1, chn)], wb)
            w0.wait()
            w1.wait()
            return 0

        lax.fori_loop(0, npair, pair, 0)

    return body


def kernel(weights, indices):
    N, F = indices.shape
    V, D = weights.shape
    rpl = 128 // D
    idx = indices.astype(jnp.int32)
    table128 = weights.reshape(V // rpl, 128)
    return _emb_gather(V, D, N, F, 4)(table128, idx // rpl, idx % rpl)


# speedup vs baseline: 1.1040x; 1.1040x over previous
import functools

import jax
import jax.numpy as jnp
from jax import lax
from jax.experimental import pallas as pl
from jax.experimental.pallas import tpu as pltpu
from jax.experimental.pallas import tpu_sc as plsc

_NC = 2
_NS = 16
_NW = _NC * _NS


def _emb_gather(V, D, N, F, chn):
    n_per_w = N // _NW
    chq = chn * F
    npair = n_per_w // (2 * chn)
    mesh = plsc.VectorSubcoreMesh(core_axis_name="c", subcore_axis_name="s")

    @functools.partial(
        pl.kernel,
        mesh=mesh,
        out_type=jax.ShapeDtypeStruct((N, F, D), jnp.float32),
        compiler_params=pltpu.CompilerParams(use_tc_tiling_on_sc=True),
        scratch_types=[
            pltpu.VMEM((chq,), jnp.int32),
            pltpu.VMEM((chq,), jnp.int32),
            pltpu.VMEM((chq,), jnp.int32),
            pltpu.VMEM((chq,), jnp.int32),
            pltpu.VMEM((chq, 128), jnp.float32),
            pltpu.VMEM((chq, 128), jnp.float32),
            pltpu.VMEM((chn, F, D), jnp.float32),
            pltpu.VMEM((chn, F, D), jnp.float32),
            pltpu.SemaphoreType.DMA,
            pltpu.SemaphoreType.DMA,
            pltpu.SemaphoreType.DMA,
            pltpu.SemaphoreType.DMA,
        ],
    )
    def body(table128, idx4, r4, out,
             ia, ib, ra, rb, rowsa, rowsb, oa, ob, ga, gb, wa, wb):
        wid = lax.axis_index("s") * _NC + lax.axis_index("c")
        qbase = wid * n_per_w * F
        nbase = wid * n_per_w

        def stage_and_fire(q0, idx_v, r_v, rows_v, gsem):
            pltpu.sync_copy(idx4.at[pl.ds(q0, chq)], idx_v)
            pltpu.sync_copy(r4.at[pl.ds(q0, chq)], r_v)
            return pltpu.async_copy(table128.at[idx_v], rows_v, gsem)

        def extract(r_v, rows_v, o_v):
            for g16 in range(chq // 16):
                rv = r_v[pl.ds(g16 * 16, 16)]
                for j in range(16):
                    q = g16 * 16 + j
                    a, f = divmod(q, F)
                    col = rv[j] * D
                    o_v[a, f, pl.ds(0, 16)] = rows_v[q, pl.ds(col, 16)]
                    o_v[a, f, pl.ds(16, 16)] = rows_v[q, pl.ds(col + 16, 16)]

        def pair(c2, _):
            q0 = qbase + c2 * (2 * chq)
            nb0 = nbase + c2 * (2 * chn)
            g0 = stage_and_fire(q0, ia, ra, rowsa, ga)
            g1 = stage_and_fire(q0 + chq, ib, rb, rowsb, gb)
            g0.wait()
            extract(ra, rowsa, oa)
            w0 = pltpu.async_copy(oa, out.at[pl.ds(nb0, chn)], wa)
            g1.wait()
            extract(rb, rowsb, ob)
            w1 = pltpu.async_copy(ob, out.at[pl.ds(nb0 + chn, chn)], wb)
            w0.wait()
            w1.wait()
            return 0

        lax.fori_loop(0, npair, pair, 0)

    return body


def kernel(weights, indices):
    N, F = indices.shape
    V, D = weights.shape
    rpl = 128 // D
    flat = indices.astype(jnp.int32).reshape(N * F)
    table128 = weights.reshape(V // rpl, 128)
    return _emb_gather(V, D, N, F, 8)(table128, flat // rpl, flat % rpl)
